# baseline (speedup 1.0000x reference)
import jax
import jax.numpy as jnp
from jax import lax
from jax.experimental import pallas as pl
from jax.experimental.pallas import tpu as pltpu
from jax.experimental.pallas import tpu_sc as plsc

VOCAB = 1000000
D = 64
B = 16384
NEG = 5

NC = 2
NS = 16
NW = NC * NS
L = 16
BPW = B // NW
CH = 64
NR = BPW // CH
GPR = CH // L
NIC = BPW // 128

TBLK = 16384


def _wbuild_body(ut_ref, vt_ref, e1_ref, e2_ref, o_ref):
    dn = (((0,), (0,)), ((), ()))
    o_ref[...] = (
        lax.dot_general(ut_ref[...], e1_ref[...], dn,
                        preferred_element_type=jnp.float32)
        + lax.dot_general(vt_ref[...], e2_ref[...], dn,
                          preferred_element_type=jnp.float32))


RPR = (2 + NEG) * CH


def _sc_body(m3, W_hbm, out_hbm, midx, rows_v, sbuf, sem0, sem1):
    wid = lax.axis_index("s") * NC + lax.axis_index("c")

    pltpu.sync_copy(m3.at[wid], midx)

    lane = lax.iota(jnp.int32, L)
    sems = (sem0, sem1)

    def fire(r):
        b = r % 2
        s = sems[b]
        cps = []
        for (o, n) in ((0, 128), (128, 128), (256, 128), (384, 64)):
            cps.append(pltpu.async_copy(
                W_hbm.at[midx.at[r, pl.ds(o, n)]],
                rows_v.at[b, pl.ds(o, n)], s))
        return cps

    pend = fire(0)
    for r in range(NR):
        nxt = fire(r + 1) if r + 1 < NR else []
        for c in pend:
            c.wait()
        pend = nxt
        b = r % 2
        bvec = jnp.full((L,), b, jnp.int32)

        def group_body(go, _):
            rows = go * L + lane

            def d_body(d, carry):
                acc_p, acc_n = carry
                du = jnp.full((L,), d, jnp.int32)
                ucol = plsc.load_gather(rows_v, [bvec, rows, du])
                vcol = plsc.load_gather(rows_v, [bvec, rows + CH, du])
                ncol = plsc.load_gather(rows_v, [bvec, rows + 2 * CH, du])
                for k in range(1, NEG):
                    ncol = ncol + plsc.load_gather(
                        rows_v, [bvec, rows + (2 + k) * CH, du])
                return acc_p + ucol * vcol, acc_n + ucol * ncol

            z = jnp.zeros((L,), jnp.float32)
            acc_p, acc_n = lax.fori_loop(0, D, d_body, (z, z), unroll=8)
            off = (r & 1) * CH + go * L
            sbuf[r >> 1, pl.ds(off, L)] = acc_p
            sbuf[NIC + (r >> 1), pl.ds(off, L)] = acc_n
            return 0

        lax.fori_loop(0, GPR, group_body, 0)

    pltpu.sync_copy(sbuf, out_hbm.at[wid])


def _loss_body(x_ref, o_ref):
    s = x_ref[:, 0:NIC, :]
    n = -x_ref[:, NIC:2 * NIC, :]

    def ls(x):
        return jnp.minimum(x, 0.0) - jnp.log1p(jnp.exp(-jnp.abs(x)))

    o_ref[...] = (-(jnp.sum(ls(s) + ls(n))) / B).reshape(1, 1)


def kernel(u, v, neg_v, U, V):
    eye = jnp.eye(D, dtype=jnp.float32)
    zer = jnp.zeros((D, D), jnp.float32)
    e1 = jnp.concatenate([eye, zer], axis=1)
    e2 = jnp.concatenate([zer, eye], axis=1)
    W = pl.pallas_call(
        _wbuild_body,
        grid=(pl.cdiv(VOCAB, TBLK),),
        in_specs=[
            pl.BlockSpec((D, TBLK), lambda j: (0, j)),
            pl.BlockSpec((D, TBLK), lambda j: (0, j)),
            pl.BlockSpec((D, 2 * D), lambda j: (0, 0)),
            pl.BlockSpec((D, 2 * D), lambda j: (0, 0)),
        ],
        out_specs=pl.BlockSpec((TBLK, 2 * D), lambda j: (j, 0)),
        out_shape=jax.ShapeDtypeStruct((VOCAB, 2 * D), jnp.float32),
    )(U.T, V.T, e1, e2)

    m3 = jnp.concatenate(
        [(2 * u).astype(jnp.int32).reshape(NW, NR, CH),
         (2 * v + 1).astype(jnp.int32).reshape(NW, NR, CH),
         (2 * neg_v + 1).astype(jnp.int32).T.reshape(NEG, NW, NR, CH)
         .transpose(1, 2, 0, 3).reshape(NW, NR, NEG * CH)],
        axis=2)

    mesh = plsc.VectorSubcoreMesh(core_axis_name="c", subcore_axis_name="s")
    packed = pl.kernel(
        _sc_body,
        out_type=jax.ShapeDtypeStruct((NW, 2 * NIC, 128), jnp.float32),
        mesh=mesh,
        compiler_params=pltpu.CompilerParams(
            needs_layout_passes=False, use_tc_tiling_on_sc=False),
        scratch_types=[
            pltpu.VMEM((NR, RPR), jnp.int32),
            pltpu.VMEM((2, RPR, D), jnp.float32),
            pltpu.VMEM((2 * NIC, 128), jnp.float32),
            pltpu.SemaphoreType.DMA,
            pltpu.SemaphoreType.DMA,
        ],
    )(m3, W.reshape(2 * VOCAB, D))

    loss = pl.pallas_call(
        _loss_body,
        out_shape=jax.ShapeDtypeStruct((1, 1), jnp.float32),
    )(packed)
    return loss[0, 0]
